# Initial kernel scaffold; baseline (speedup 1.0000x reference)
import jax
import jax.numpy as jnp
from jax.experimental import pallas as pl


def kernel(node_weights, node_prev_features, self_idx, neighbor_idx, gate_W1, gate_b1, gate_W2, gate_b2, msg_W1, msg_b1, msg_W2, msg_b2, pow_param):
    raise NotImplementedError("write your pallas kernel here")


# speedup vs baseline: 3.5802x; 3.5802x over previous
import jax
import jax.numpy as jnp
from jax import lax
from jax.experimental import pallas as pl
from jax.experimental.pallas import tpu as pltpu
from jax.experimental.pallas import tpu_sc as plsc

N = 10000
E = 320000
D = 128
H = 256

NC = 2
NS = 16
NW = NC * NS
EPT = E // NW
CH = 80
NCHUNK = EPT // CH
AUGW = 144
ROWS_PER_TILE = N // NS
ZROWS = 125

_vector_mesh = plsc.VectorSubcoreMesh(core_axis_name="c", subcore_axis_name="s")
_sc_params = pltpu.CompilerParams(use_tc_tiling_on_sc=False)


def _sc_gather(features, aug, idx_self, idx_nbr):
    @pl.kernel(
        out_type=[
            jax.ShapeDtypeStruct((E, D), jnp.float32),
            jax.ShapeDtypeStruct((E, AUGW), jnp.float32),
        ],
        mesh=_vector_mesh,
        scratch_types=[
            pltpu.VMEM((1, CH), jnp.int32),
            pltpu.VMEM((1, CH), jnp.int32),
            pltpu.VMEM((CH, D), jnp.float32),
            pltpu.VMEM((CH, AUGW), jnp.float32),
            pltpu.SemaphoreType.DMA,
            pltpu.SemaphoreType.DMA,
        ],
        compiler_params=_sc_params,
    )
    def kern(feat_hbm, aug_hbm, idxs_hbm, idxn_hbm, outs_hbm, outn_hbm,
             idxs_v, idxn_v, bufs, bufn, sem1, sem2):
        c = lax.axis_index("c")
        s = lax.axis_index("s")
        base = (c * NS + s) * EPT

        @pl.loop(0, NCHUNK)
        def _(k):
            off = base + k * CH
            pltpu.sync_copy(idxs_hbm.at[pl.ds(off, CH)], idxs_v.at[0])
            pltpu.sync_copy(idxn_hbm.at[pl.ds(off, CH)], idxn_v.at[0])
            g1 = pltpu.async_copy(feat_hbm.at[idxs_v.at[0]], bufs, sem1)
            g2 = pltpu.async_copy(aug_hbm.at[idxn_v.at[0]], bufn, sem2)
            g1.wait()
            g2.wait()
            pltpu.sync_copy(bufs, outs_hbm.at[pl.ds(off, CH)])
            pltpu.sync_copy(bufn, outn_hbm.at[pl.ds(off, CH)])

    return kern(features, aug, idx_self, idx_nbr)


def _tc_dense(rows_self, rows_nbr, W1cat, b1cat, W2blk, b2cat, powp):
    BB = 512
    grid = E // BB

    def body(self_ref, nbr_ref, w1_ref, b1_ref, w2_ref, b2_ref, p_ref, out_ref):
        x = jnp.concatenate([self_ref[...], nbr_ref[:, :D]], axis=1)
        h = jnp.dot(x, w1_ref[...], preferred_element_type=jnp.float32)
        h = h + b1_ref[...]
        h = jnp.where(h > 0, h, 0.01 * h)
        gm = jnp.dot(h, w2_ref[...], preferred_element_type=jnp.float32)
        gm = gm + b2_ref[...]
        g = gm[:, 0:1]
        m = gm[:, 1:129]
        w = nbr_ref[:, D:D + 1]
        p = p_ref[0, 0]
        e = (w ** p) * jnp.exp(g)
        out_ref[...] = jnp.concatenate(
            [e * m, e, jnp.zeros((BB, AUGW - D - 1), jnp.float32)], axis=1)

    return pl.pallas_call(
        body,
        grid=(grid,),
        in_specs=[
            pl.BlockSpec((BB, D), lambda i: (i, 0)),
            pl.BlockSpec((BB, AUGW), lambda i: (i, 0)),
            pl.BlockSpec((2 * D, 2 * H), lambda i: (0, 0)),
            pl.BlockSpec((1, 2 * H), lambda i: (0, 0)),
            pl.BlockSpec((2 * H, D + 1), lambda i: (0, 0)),
            pl.BlockSpec((1, D + 1), lambda i: (0, 0)),
            pl.BlockSpec((1, 1), lambda i: (0, 0)),
        ],
        out_specs=pl.BlockSpec((BB, AUGW), lambda i: (i, 0)),
        out_shape=jax.ShapeDtypeStruct((E, AUGW), jnp.float32),
    )(rows_self, rows_nbr, W1cat, b1cat, W2blk, b2cat, powp)


def _sc_scatter(contrib, idx_self):
    @pl.kernel(
        out_type=jax.ShapeDtypeStruct((NC, N, AUGW), jnp.float32),
        mesh=_vector_mesh,
        scratch_types=[
            pltpu.VMEM_SHARED((N, AUGW), jnp.float32),
            pltpu.VMEM((ZROWS, AUGW), jnp.float32),
            pltpu.VMEM((CH, AUGW), jnp.float32),
            pltpu.VMEM((1, CH), jnp.int32),
        ],
        compiler_params=_sc_params,
    )
    def kern(contrib_hbm, idx_hbm, out_hbm, shared, zbuf, cbuf, idx_v):
        c = lax.axis_index("c")
        s = lax.axis_index("s")

        @pl.loop(0, ZROWS)
        def _(r):
            @pl.loop(0, AUGW // 16)
            def _(ct):
                zbuf[r, pl.ds(ct * 16, 16)] = jnp.zeros((16,), jnp.float32)

        @pl.loop(0, ROWS_PER_TILE // ZROWS)
        def _(j):
            pltpu.sync_copy(zbuf, shared.at[pl.ds(s * ROWS_PER_TILE + j * ZROWS, ZROWS)])

        plsc.subcore_barrier()

        base = (c * NS + s) * EPT

        @pl.loop(0, NCHUNK)
        def _(k):
            off = base + k * CH
            pltpu.sync_copy(idx_hbm.at[pl.ds(off, CH)], idx_v.at[0])
            pltpu.sync_copy(contrib_hbm.at[pl.ds(off, CH)], cbuf)
            pltpu.sync_copy(cbuf, shared.at[idx_v.at[0]], add=True)

        plsc.subcore_barrier()

        @pl.loop(0, ROWS_PER_TILE // ZROWS)
        def _(j):
            row = s * ROWS_PER_TILE + j * ZROWS
            pltpu.sync_copy(shared.at[pl.ds(row, ZROWS)],
                            out_hbm.at[c].at[pl.ds(row, ZROWS)])

    return kern(contrib, idx_self)


def _tc_finalize(partials, features):
    BN = 2000

    def body(p_ref, f_ref, o_ref):
        num = p_ref[0, :, :D] + p_ref[1, :, :D]
        den = p_ref[0, :, D:D + 1] + p_ref[1, :, D:D + 1]
        o_ref[...] = num / (den + 1e-10) + f_ref[...]

    return pl.pallas_call(
        body,
        grid=(N // BN,),
        in_specs=[
            pl.BlockSpec((NC, BN, AUGW), lambda i: (0, i, 0)),
            pl.BlockSpec((BN, D), lambda i: (i, 0)),
        ],
        out_specs=pl.BlockSpec((BN, D), lambda i: (i, 0)),
        out_shape=jax.ShapeDtypeStruct((N, D), jnp.float32),
    )(partials, features)


def kernel(node_weights, node_prev_features, self_idx, neighbor_idx,
           gate_W1, gate_b1, gate_W2, gate_b2,
           msg_W1, msg_b1, msg_W2, msg_b2, pow_param):
    idx_self = self_idx.astype(jnp.int32)
    idx_nbr = neighbor_idx.astype(jnp.int32)
    feats = node_prev_features.astype(jnp.float32)

    aug = jnp.concatenate(
        [feats, node_weights.astype(jnp.float32),
         jnp.zeros((N, AUGW - D - 1), jnp.float32)], axis=1)

    W1cat = jnp.concatenate([gate_W1, msg_W1], axis=1)
    b1cat = jnp.concatenate([gate_b1, msg_b1])[None, :]
    W2blk = jnp.zeros((2 * H, D + 1), jnp.float32)
    W2blk = W2blk.at[:H, 0:1].set(gate_W2)
    W2blk = W2blk.at[H:, 1:].set(msg_W2)
    b2cat = jnp.concatenate([gate_b2, msg_b2])[None, :]
    powp = pow_param.reshape(1, 1).astype(jnp.float32)

    rows_self, rows_nbr = _sc_gather(feats, aug, idx_self, idx_nbr)
    contrib = _tc_dense(rows_self, rows_nbr, W1cat, b1cat, W2blk, b2cat, powp)
    partials = _sc_scatter(contrib, idx_self)
    return _tc_finalize(partials, feats)
